# Initial kernel scaffold; baseline (speedup 1.0000x reference)
import jax
import jax.numpy as jnp
from jax.experimental import pallas as pl


def kernel(x, spatial_attention, conv_weight, conv_bias):
    raise NotImplementedError("write your pallas kernel here")


# speedup vs baseline: 11.1077x; 11.1077x over previous
import functools

import jax
import jax.numpy as jnp
from jax import lax
from jax.experimental import pallas as pl
from jax.experimental.pallas import tpu as pltpu
from jax.experimental.pallas import tpu_sc as plsc

B, C, H, W = 16, 96, 224, 224
HW = H * W
R = B * C
M = 8
NC = HW // M
K = 1024
SUB = HW // 128
NSURV = 1280
NCAND = NSURV * M
NWIN = 1088
L = 16

_U32 = jnp.uint32


def _tc_cmax_kernel(x_ref, att_ref, out_ref):
    a = x_ref[0, 0] * att_ref[0]
    m = a
    for s in (1, 2, 4):
        m = jnp.maximum(m, pltpu.roll(m, 128 - s, 1))
    lidx = lax.broadcasted_iota(jnp.int32, (128, 16), 0)
    cidx = lax.broadcasted_iota(jnp.int32, (128, 16), 1)
    sel = (lidx == cidx * 8).astype(jnp.float32)
    cm = lax.dot_general(m, sel, (((1,), (0,)), ((), ())),
                         preferred_element_type=jnp.float32)
    bits = lax.bitcast_convert_type(cm, _U32)
    flip = jnp.where(bits >= _U32(0x80000000), _U32(0xFFFFFFFF),
                     _U32(0x80000000))
    out_ref[0, 0] = bits ^ flip


def _key16(v):
    bits = plsc.bitcast(v, _U32)
    flip = jnp.where(bits >= _U32(0x80000000), _U32(0xFFFFFFFF),
                     _U32(0x80000000))
    return bits ^ flip


def _unkey16(k):
    bits = jnp.where(k >= _U32(0x80000000), k ^ _U32(0x80000000), ~k)
    return plsc.bitcast(bits, jnp.float32)


def _sc_body(x_hbm, att_hbm, ck_hbm, w_hbm, bias_hbm, locs_hbm, red_hbm,
             ck_row, hist, offs, bnd_a, bnd_b, surv_x,
             candx, canda, candk, wkk, wkp, skk, skp,
             wrow, biasv, outloc, outred, sem1, sem2):
    lanes = lax.iota(jnp.int32, L)
    ones = jnp.ones((L,), jnp.int32)
    zeros_i = jnp.zeros((L,), jnp.int32)

    def zero_hist(href):
        def zh(i, _):
            href[pl.ds(i * L, L)] = zeros_i
            return 0
        lax.fori_loop(0, 256 // L, zh, 0)

    def digits(k, shift):
        return lax.shift_right_logical(k, jnp.int32(shift)) & jnp.int32(0xFF)

    def hist_ref_pass(src, n, shift):
        zero_hist(hist)
        nv = (n + (L - 1)) // L

        def hp(i, _):
            v = src[pl.ds(i * L, L)]
            valid = (lanes + i * L) < n
            plsc.addupdate_scatter(hist, [digits(v, shift)], ones, mask=valid)
            return 0
        lax.fori_loop(0, nv, hp, 0)

    def scan_hist(krem):
        def sp(j, carry):
            acc, found, beta, kr = carry
            hv = hist[pl.ds(240 - L * j, L)]
            rv = lax.rev(hv, (0,))
            cs = plsc.cumsum(rv) + acc
            cross = cs >= krem
            anyc = jnp.max(cross.astype(jnp.int32))
            idx = jnp.max(plsc.all_reduce_ffs(cross))
            csv = jnp.max(jnp.where(lanes == idx, cs, 0))
            hvv = jnp.max(jnp.where(lanes == idx, rv, 0))
            nbeta = 255 - L * j - idx
            take = (found == 0) & (anyc == 1)
            beta = jnp.where(take, nbeta, beta)
            kr = jnp.where(take, krem - (csv - hvv), kr)
            found = jnp.where(anyc == 1, 1, found)
            return jnp.max(cs), found, beta, kr
        z = jnp.int32(0)
        _, _, beta, kr = lax.fori_loop(
            0, 256 // L, sp, (z, z, z, jnp.int32(krem)))
        return beta, kr

    def compact_eq(src, n, shift, beta, dst):
        nv = (n + (L - 1)) // L

        def cp(i, cnt):
            v = src[pl.ds(i * L, L)]
            valid = (lanes + i * L) < n
            m = valid & (digits(v, shift) == beta)
            cs = plsc.cumsum(m.astype(jnp.int32))
            plsc.store_scatter(dst, [cnt + cs - 1], v, mask=m)
            return cnt + jnp.max(cs)
        return lax.fori_loop(0, nv, cp, jnp.int32(0))

    def radix_select(src, n, krem):
        hist_ref_pass(src, n, 24)
        b1, krem = scan_hist(krem)
        n1 = compact_eq(src, n, 24, b1, bnd_a)
        hist_ref_pass(bnd_a, n1, 16)
        b2, krem = scan_hist(krem)
        n2 = compact_eq(bnd_a, n1, 16, b2, bnd_b)
        hist_ref_pass(bnd_b, n2, 8)
        b3, krem = scan_hist(krem)
        n3 = compact_eq(bnd_b, n2, 8, b3, bnd_a)
        hist_ref_pass(bnd_a, n3, 0)
        b4, _ = scan_hist(krem)
        bu = lambda b, s: lax.shift_left(
            lax.convert_element_type(b, _U32), _U32(s))
        return bu(b1, 24) | bu(b2, 16) | bu(b3, 8) | bu(b4, 0)

    def radix_sort_pass(srck, srcp, dstk, dstp, shift):
        zero_hist(hist)

        def hp(i, _):
            d = digits(srck[pl.ds(i * L, L)], shift)
            plsc.addupdate_scatter(hist, [d], ones)
            return 0
        lax.fori_loop(0, NWIN // L, hp, 0)

        def op(j, acc):
            hv = hist[pl.ds(L * j, L)]
            cs = plsc.cumsum(hv) + acc
            offs[pl.ds(L * j, L)] = cs - hv
            return jnp.max(cs)
        lax.fori_loop(0, 256 // L, op, jnp.int32(0))

        def mp(i, _):
            kv = srck[pl.ds(i * L, L)]
            pv = srcp[pl.ds(i * L, L)]
            d = digits(kv, shift)
            cnt, _last = plsc.scan_count(plsc.bitcast(d, _U32))
            rank = plsc.bitcast(cnt, jnp.int32) - 1
            dest = plsc.load_gather(offs, [d]) + rank
            plsc.store_scatter(dstk, [dest], kv)
            plsc.store_scatter(dstp, [dest], pv)
            plsc.addupdate_scatter(offs, [d], ones)
            return 0
        lax.fori_loop(0, NWIN // L, mp, 0)

    wid = lax.axis_index("s") * 2 + lax.axis_index("c")
    nch = C // 32

    def ch_body(ci, _):
        ch = wid * nch + ci
        pltpu.sync_copy(w_hbm.at[ch], wrow)
        pltpu.sync_copy(bias_hbm.at[ch], biasv)

        def row_body(b, _):
            r = b * C + ch
            pltpu.sync_copy(ck_hbm.at[r], ck_row)

            t_c = radix_select(ck_row, NC, K)

            def pf(i, _):
                surv_x[pl.ds(i * L, L)] = lanes + i * L
                return 0
            lax.fori_loop(0, NSURV // L, pf, 0)

            def sv(i, cnt):
                v = plsc.bitcast(ck_row[pl.ds(i * L, L)], _U32)
                m = v >= t_c
                cs = plsc.cumsum(m.astype(jnp.int32))
                idx = cnt + cs - 1
                m2 = m & (idx < NSURV)
                plsc.store_scatter(surv_x, [idx], lanes + i * L, mask=m2)
                return cnt + jnp.max(cs)
            n_s = lax.fori_loop(0, NC // L, sv, jnp.int32(0))
            n_s = jnp.minimum(n_s, NSURV)
            nt = (n_s + (L - 1)) // L

            def gl(t, _):
                ids = surv_x[pl.ds(t * L, L)]
                pltpu.make_async_copy(
                    x_hbm.at[ids + r * NC],
                    candx.at[pl.ds(t * L, L)], sem1).start()
                pltpu.make_async_copy(
                    att_hbm.at[ids + ch * NC],
                    canda.at[pl.ds(t * L, L)], sem2).start()
                return 0
            lax.fori_loop(0, nt, gl, 0)

            def gw(t, _):
                pltpu.make_async_copy(
                    x_hbm.at[pl.ds(0, L)],
                    candx.at[pl.ds(t * L, L)], sem1).wait()
                pltpu.make_async_copy(
                    att_hbm.at[pl.ds(0, L)],
                    canda.at[pl.ds(t * L, L)], sem2).wait()
                return 0
            lax.fori_loop(0, nt, gw, 0)

            n_g = n_s * M

            def kp(i, _):
                j = lanes + i * L
                jc = j // M
                jr = j - jc * M
                xv = plsc.load_gather(candx, [jc, jr])
                av = plsc.load_gather(canda, [jc, jr])
                valid = j < n_g
                kk = jnp.where(valid, _key16(xv * av), _U32(0))
                candk[pl.ds(i * L, L)] = plsc.bitcast(kk, jnp.int32)
                return 0
            lax.fori_loop(0, (n_g + (L - 1)) // L, kp, 0)

            e_t = radix_select(candk, n_g, K)

            def wf(i, _):
                wkk[pl.ds(i * L, L)] = jnp.full((L,), -1, jnp.int32)
                return 0
            lax.fori_loop(0, NWIN // L, wf, 0)

            def wp(i, cnt):
                kk = plsc.bitcast(candk[pl.ds(i * L, L)], _U32)
                j = lanes + i * L
                valid = j < n_g
                m = valid & (kk >= e_t)
                cs = plsc.cumsum(m.astype(jnp.int32))
                idx = cnt + cs - 1
                m2 = m & (idx < NWIN)
                jc = j // M
                gx = plsc.load_gather(surv_x, [jc])
                pos = gx * M + (j - jc * M)
                plsc.store_scatter(wkk, [idx], plsc.bitcast(~kk, jnp.int32), mask=m2)
                plsc.store_scatter(wkp, [idx], pos, mask=m2)
                return cnt + jnp.max(cs)
            lax.fori_loop(0, (n_g + (L - 1)) // L, wp, jnp.int32(0))

            radix_sort_pass(wkk, wkp, skk, skp, 0)
            radix_sort_pass(skk, skp, wkk, wkp, 8)
            radix_sort_pass(wkk, wkp, skk, skp, 16)
            radix_sort_pass(skk, skp, wkk, wkp, 24)

            def rp(i, acc):
                inv = plsc.bitcast(wkk[pl.ds(i * L, L)], _U32)
                p = wkp[pl.ds(i * L, L)]
                val = _unkey16(~inv)
                wv = wrow[pl.ds(i * L, L)]
                outloc[pl.ds(i * L, L)] = lax.convert_element_type(
                    p, jnp.float32)
                return acc + val * wv
            acc = lax.fori_loop(0, K // L, rp, jnp.zeros((L,), jnp.float32))
            red = jnp.sum(acc) + jnp.max(biasv[...])
            outred[...] = jnp.broadcast_to(red, (L,))
            pltpu.sync_copy(outloc, locs_hbm.at[r])
            pltpu.sync_copy(outred, red_hbm.at[r])
            return 0
        lax.fori_loop(0, B, row_body, 0)
        return 0
    lax.fori_loop(0, nch, ch_body, 0)


def kernel(x, spatial_attention, conv_weight, conv_bias):
    x4 = x.reshape(B, C, SUB, 128)
    att4 = spatial_attention.reshape(C, SUB, 128)
    ck = pl.pallas_call(
        _tc_cmax_kernel,
        grid=(C, B),
        in_specs=[
            pl.BlockSpec((1, 1, SUB, 128), lambda c, b: (b, c, 0, 0)),
            pl.BlockSpec((1, SUB, 128), lambda c, b: (c, 0, 0)),
        ],
        out_specs=pl.BlockSpec((1, 1, SUB, 16), lambda c, b: (b, c, 0, 0)),
        out_shape=jax.ShapeDtypeStruct((B, C, SUB, 16), _U32),
    )(x4, att4)

    mesh = plsc.VectorSubcoreMesh(core_axis_name="c", subcore_axis_name="s")
    sc = functools.partial(
        pl.kernel,
        out_type=(jax.ShapeDtypeStruct((R, K), jnp.float32),
                  jax.ShapeDtypeStruct((R, L), jnp.float32)),
        mesh=mesh,
        scratch_types=[
            pltpu.VMEM((NC,), jnp.int32),
            pltpu.VMEM((256,), jnp.int32),
            pltpu.VMEM((256,), jnp.int32),
            pltpu.VMEM((NCAND,), jnp.int32),
            pltpu.VMEM((NCAND,), jnp.int32),
            pltpu.VMEM((NSURV,), jnp.int32),
            pltpu.VMEM((NSURV, M), jnp.float32),
            pltpu.VMEM((NSURV, M), jnp.float32),
            pltpu.VMEM((NCAND,), jnp.int32),
            pltpu.VMEM((NWIN,), jnp.int32),
            pltpu.VMEM((NWIN,), jnp.int32),
            pltpu.VMEM((NWIN,), jnp.int32),
            pltpu.VMEM((NWIN,), jnp.int32),
            pltpu.VMEM((K,), jnp.float32),
            pltpu.VMEM((L,), jnp.float32),
            pltpu.VMEM((K,), jnp.float32),
            pltpu.VMEM((L,), jnp.float32),
            pltpu.SemaphoreType.DMA,
            pltpu.SemaphoreType.DMA,
        ],
        compiler_params=pltpu.CompilerParams(
            needs_layout_passes=False, use_tc_tiling_on_sc=False),
    )(_sc_body)

    x3 = x.reshape(R * NC, M)
    att3 = spatial_attention.reshape(C * NC, M)
    ck2 = lax.bitcast_convert_type(ck, jnp.int32).reshape(R, NC)
    w2 = conv_weight.reshape(C, K)
    bias2 = jnp.broadcast_to(conv_bias.reshape(C, 1), (C, L))
    locs, red = sc(x3, att3, ck2, w2, bias2)
    return (red[:, 0].reshape(B, C), locs.reshape(B, C, K))


# speedup vs baseline: 12.8759x; 1.1592x over previous
import functools

import jax
import jax.numpy as jnp
from jax import lax
from jax.experimental import pallas as pl
from jax.experimental.pallas import tpu as pltpu
from jax.experimental.pallas import tpu_sc as plsc

B, C, H, W = 16, 96, 224, 224
HW = H * W
R = B * C
M = 8
NC = HW // M
K = 1024
SUB = HW // 128
NSURV = 1280
NCAND = NSURV * M
NWIN = 1088
L = 16

_U32 = jnp.uint32


def _tc_cmax_kernel(x_ref, att_ref, out_ref):
    lidx = lax.broadcasted_iota(jnp.int32, (128, 16), 0)
    cidx = lax.broadcasted_iota(jnp.int32, (128, 16), 1)
    sel = (lidx == cidx * 8).astype(jnp.float32)
    att = att_ref[0]
    for q in range(4):
        a = x_ref[q, 0] * att
        m = a
        for s in (1, 2, 4):
            m = jnp.maximum(m, pltpu.roll(m, 128 - s, 1))
        cm = lax.dot_general(m, sel, (((1,), (0,)), ((), ())),
                             preferred_element_type=jnp.float32)
        bits = lax.bitcast_convert_type(cm, _U32)
        flip = jnp.where(bits >= _U32(0x80000000), _U32(0xFFFFFFFF),
                         _U32(0x80000000))
        out_ref[q, 0] = bits ^ flip


def _key16(v):
    bits = plsc.bitcast(v, _U32)
    flip = jnp.where(bits >= _U32(0x80000000), _U32(0xFFFFFFFF),
                     _U32(0x80000000))
    return bits ^ flip


def _unkey16(k):
    bits = jnp.where(k >= _U32(0x80000000), k ^ _U32(0x80000000), ~k)
    return plsc.bitcast(bits, jnp.float32)


def _sc_body(x_hbm, att_hbm, ck_hbm, w_hbm, bias_hbm, locs_hbm, red_hbm,
             ck_row, hist, offs, bnd_a, bnd_b, surv_x, surv_a,
             candx, canda, candk, wkk, wkp, skk, skp,
             wrow, biasv, outloc, outred, sem1, sem2):
    lanes = lax.iota(jnp.int32, L)
    ones = jnp.ones((L,), jnp.int32)
    zeros_i = jnp.zeros((L,), jnp.int32)

    def zero_hist(href):
        def zh(i, _):
            href[pl.ds(i * L, L)] = zeros_i
            return 0
        lax.fori_loop(0, 256 // L, zh, 0)

    def digits(k, shift):
        return lax.shift_right_logical(k, jnp.int32(shift)) & jnp.int32(0xFF)

    def hist_ref_pass(src, n, shift):
        zero_hist(hist)
        nb = (n + (4 * L - 1)) // (4 * L)

        def hp(i, _):
            base = i * (4 * L)
            for u in range(4):
                v = src[pl.ds(base + u * L, L)]
                valid = (lanes + (base + u * L)) < n
                plsc.addupdate_scatter(hist, [digits(v, shift)], ones,
                                       mask=valid)
            return 0
        lax.fori_loop(0, nb, hp, 0)

    def scan_hist(krem):
        def cond(carry):
            j, acc, found, beta, kr = carry
            return (found == 0) & (j < 256 // L)

        def sp(carry):
            j, acc, found, beta, kr = carry
            hv = hist[pl.ds(240 - L * j, L)]
            rv = lax.rev(hv, (0,))
            cs = plsc.cumsum(rv) + acc
            cross = cs >= krem
            anyc = jnp.max(cross.astype(jnp.int32))
            idx = jnp.max(plsc.all_reduce_ffs(cross))
            csv = jnp.max(jnp.where(lanes == idx, cs, 0))
            hvv = jnp.max(jnp.where(lanes == idx, rv, 0))
            nbeta = 255 - L * j - idx
            take = anyc == 1
            beta = jnp.where(take, nbeta, beta)
            kr = jnp.where(take, krem - (csv - hvv), kr)
            found = jnp.where(take, 1, found)
            return j + 1, jnp.max(cs), found, beta, kr
        z = jnp.int32(0)
        _, _, _, beta, kr = lax.while_loop(
            cond, sp, (z, z, z, z, jnp.int32(krem)))
        return beta, kr

    def compact_eq(src, n, shift, beta, dst):
        nb = (n + (4 * L - 1)) // (4 * L)

        def cp(i, cnt):
            base = i * (4 * L)
            for u in range(4):
                v = src[pl.ds(base + u * L, L)]
                valid = (lanes + (base + u * L)) < n
                m = valid & (digits(v, shift) == beta)
                cs = plsc.cumsum(m.astype(jnp.int32))
                plsc.store_scatter(dst, [cnt + cs - 1], v, mask=m)
                cnt = cnt + plsc.all_reduce_population_count(m)
            return cnt
        cntv = lax.fori_loop(0, nb, cp, jnp.zeros((L,), jnp.int32))
        return jnp.max(cntv)

    def radix_select(src, n, krem):
        hist_ref_pass(src, n, 24)
        b1, krem = scan_hist(krem)
        n1 = compact_eq(src, n, 24, b1, bnd_a)
        hist_ref_pass(bnd_a, n1, 16)
        b2, krem = scan_hist(krem)
        n2 = compact_eq(bnd_a, n1, 16, b2, bnd_b)
        hist_ref_pass(bnd_b, n2, 8)
        b3, krem = scan_hist(krem)
        n3 = compact_eq(bnd_b, n2, 8, b3, bnd_a)
        hist_ref_pass(bnd_a, n3, 0)
        b4, _ = scan_hist(krem)
        bu = lambda b, s: lax.shift_left(
            lax.convert_element_type(b, _U32), _U32(s))
        return bu(b1, 24) | bu(b2, 16) | bu(b3, 8) | bu(b4, 0)

    def radix_sort_pass(srck, srcp, dstk, dstp, shift):
        zero_hist(hist)

        def hp(i, _):
            base = i * (4 * L)
            for u in range(4):
                d = digits(srck[pl.ds(base + u * L, L)], shift)
                plsc.addupdate_scatter(hist, [d], ones)
            return 0
        lax.fori_loop(0, NWIN // (4 * L), hp, 0)

        def op(j, acc):
            base = j * (4 * L)
            for u in range(4):
                hv = hist[pl.ds(base + u * L, L)]
                cs = plsc.cumsum(hv) + acc
                offs[pl.ds(base + u * L, L)] = cs - hv
                acc = jnp.max(cs)
            return acc
        lax.fori_loop(0, 256 // (4 * L), op, jnp.int32(0))

        def mp(i, _):
            base = i * (4 * L)
            for u in range(4):
                kv = srck[pl.ds(base + u * L, L)]
                pv = srcp[pl.ds(base + u * L, L)]
                d = digits(kv, shift)
                cnt, _last = plsc.scan_count(plsc.bitcast(d, _U32))
                rank = plsc.bitcast(cnt, jnp.int32) - 1
                dest = plsc.load_gather(offs, [d]) + rank
                plsc.store_scatter(dstk, [dest], kv)
                plsc.store_scatter(dstp, [dest], pv)
                plsc.addupdate_scatter(offs, [d], ones)
            return 0
        lax.fori_loop(0, NWIN // (4 * L), mp, 0)

    wid = lax.axis_index("s") * 2 + lax.axis_index("c")
    nch = C // 32

    def ch_body(ci, _):
        ch = wid * nch + ci
        pltpu.sync_copy(w_hbm.at[ch], wrow)
        pltpu.sync_copy(bias_hbm.at[ch], biasv)

        def row_body(b, _):
            r = b * C + ch
            pltpu.sync_copy(ck_hbm.at[r], ck_row)

            t_c = radix_select(ck_row, NC, K)

            def pf(i, _):
                base = i * (4 * L)
                for u in range(4):
                    ids = lanes + base + u * L
                    surv_x[pl.ds(base + u * L, L)] = ids + r * NC
                    surv_a[pl.ds(base + u * L, L)] = ids + ch * NC
                return 0
            lax.fori_loop(0, NSURV // (4 * L), pf, 0)

            def sv(i, cnt):
                base = i * (4 * L)
                for u in range(4):
                    v = plsc.bitcast(ck_row[pl.ds(base + u * L, L)], _U32)
                    m = v >= t_c
                    cs = plsc.cumsum(m.astype(jnp.int32))
                    idx = cnt + cs - 1
                    m2 = m & (idx < NSURV)
                    ids = lanes + base + u * L
                    plsc.store_scatter(surv_x, [idx], ids + r * NC, mask=m2)
                    plsc.store_scatter(surv_a, [idx], ids + ch * NC, mask=m2)
                    cnt = cnt + plsc.all_reduce_population_count(m)
                return cnt
            n_s = jnp.max(lax.fori_loop(0, NC // (4 * L), sv,
                                        jnp.zeros((L,), jnp.int32)))
            n_s = jnp.minimum(n_s, NSURV)

            G = 128
            ntb = (n_s + (G - 1)) // G

            def gl(t, _):
                pltpu.make_async_copy(
                    x_hbm.at[surv_x.at[pl.ds(t * G, G)]],
                    candx.at[pl.ds(t * G, G)], sem1).start()
                pltpu.make_async_copy(
                    att_hbm.at[surv_a.at[pl.ds(t * G, G)]],
                    canda.at[pl.ds(t * G, G)], sem2).start()
                return 0
            lax.fori_loop(0, ntb, gl, 0)

            def gw(t, _):
                pltpu.make_async_copy(
                    x_hbm.at[pl.ds(0, G)],
                    candx.at[pl.ds(t * G, G)], sem1).wait()
                pltpu.make_async_copy(
                    att_hbm.at[pl.ds(0, G)],
                    canda.at[pl.ds(t * G, G)], sem2).wait()
                return 0
            lax.fori_loop(0, ntb, gw, 0)

            n_g = n_s * M

            def kp(i, _):
                base = i * (4 * L)
                for u in range(4):
                    j = lanes + base + u * L
                    jc = j // M
                    jr = j - jc * M
                    xv = plsc.load_gather(candx, [jc, jr])
                    av = plsc.load_gather(canda, [jc, jr])
                    valid = j < n_g
                    kk = jnp.where(valid, _key16(xv * av), _U32(0))
                    candk[pl.ds(base + u * L, L)] = plsc.bitcast(
                        kk, jnp.int32)
                return 0
            lax.fori_loop(0, (n_g + (4 * L - 1)) // (4 * L), kp, 0)

            e_t = radix_select(candk, n_g, K)

            neg1 = jnp.full((L,), -1, jnp.int32)

            def wf(i, _):
                base = i * (4 * L)
                for u in range(4):
                    wkk[pl.ds(base + u * L, L)] = neg1
                return 0
            lax.fori_loop(0, NWIN // (4 * L), wf, 0)

            def wp(i, cnt):
                base = i * (4 * L)
                for u in range(4):
                    j = lanes + base + u * L
                    kk = plsc.bitcast(candk[pl.ds(base + u * L, L)], _U32)
                    valid = j < n_g
                    m = valid & (kk >= e_t)
                    cs = plsc.cumsum(m.astype(jnp.int32))
                    idx = cnt + cs - 1
                    m2 = m & (idx < NWIN)
                    jc = j // M
                    gx = plsc.load_gather(surv_x, [jc]) - r * NC
                    pos = gx * M + (j - jc * M)
                    plsc.store_scatter(wkk, [idx],
                                       plsc.bitcast(~kk, jnp.int32), mask=m2)
                    plsc.store_scatter(wkp, [idx], pos, mask=m2)
                    cnt = cnt + plsc.all_reduce_population_count(m)
                return cnt
            lax.fori_loop(0, (n_g + (4 * L - 1)) // (4 * L), wp,
                          jnp.zeros((L,), jnp.int32))

            radix_sort_pass(wkk, wkp, skk, skp, 0)
            radix_sort_pass(skk, skp, wkk, wkp, 8)
            radix_sort_pass(wkk, wkp, skk, skp, 16)
            radix_sort_pass(skk, skp, wkk, wkp, 24)

            def rp(i, acc):
                base = i * (4 * L)
                for u in range(4):
                    inv = plsc.bitcast(wkk[pl.ds(base + u * L, L)], _U32)
                    p = wkp[pl.ds(base + u * L, L)]
                    val = _unkey16(~inv)
                    wv = wrow[pl.ds(base + u * L, L)]
                    outloc[pl.ds(base + u * L, L)] = lax.convert_element_type(
                        p, jnp.float32)
                    acc = acc + val * wv
                return acc
            acc = lax.fori_loop(0, K // (4 * L), rp,
                                jnp.zeros((L,), jnp.float32))
            red = jnp.sum(acc) + jnp.max(biasv[...])
            outred[...] = jnp.broadcast_to(red, (L,))
            pltpu.sync_copy(outloc, locs_hbm.at[r])
            pltpu.sync_copy(outred, red_hbm.at[r])
            return 0
        lax.fori_loop(0, B, row_body, 0)
        return 0
    lax.fori_loop(0, nch, ch_body, 0)


def kernel(x, spatial_attention, conv_weight, conv_bias):
    x4 = x.reshape(B, C, SUB, 128)
    att4 = spatial_attention.reshape(C, SUB, 128)
    ck = pl.pallas_call(
        _tc_cmax_kernel,
        grid=(C, B // 4),
        in_specs=[
            pl.BlockSpec((4, 1, SUB, 128), lambda c, b: (b, c, 0, 0)),
            pl.BlockSpec((1, SUB, 128), lambda c, b: (c, 0, 0)),
        ],
        out_specs=pl.BlockSpec((4, 1, SUB, 16), lambda c, b: (b, c, 0, 0)),
        out_shape=jax.ShapeDtypeStruct((B, C, SUB, 16), _U32),
    )(x4, att4)

    mesh = plsc.VectorSubcoreMesh(core_axis_name="c", subcore_axis_name="s")
    sc = functools.partial(
        pl.kernel,
        out_type=(jax.ShapeDtypeStruct((R, K), jnp.float32),
                  jax.ShapeDtypeStruct((R, L), jnp.float32)),
        mesh=mesh,
        scratch_types=[
            pltpu.VMEM((NC,), jnp.int32),
            pltpu.VMEM((256,), jnp.int32),
            pltpu.VMEM((256,), jnp.int32),
            pltpu.VMEM((NCAND,), jnp.int32),
            pltpu.VMEM((NCAND,), jnp.int32),
            pltpu.VMEM((NSURV,), jnp.int32),
            pltpu.VMEM((NSURV,), jnp.int32),
            pltpu.VMEM((NSURV, M), jnp.float32),
            pltpu.VMEM((NSURV, M), jnp.float32),
            pltpu.VMEM((NCAND,), jnp.int32),
            pltpu.VMEM((NWIN,), jnp.int32),
            pltpu.VMEM((NWIN,), jnp.int32),
            pltpu.VMEM((NWIN,), jnp.int32),
            pltpu.VMEM((NWIN,), jnp.int32),
            pltpu.VMEM((K,), jnp.float32),
            pltpu.VMEM((L,), jnp.float32),
            pltpu.VMEM((K,), jnp.float32),
            pltpu.VMEM((L,), jnp.float32),
            pltpu.SemaphoreType.DMA,
            pltpu.SemaphoreType.DMA,
        ],
        compiler_params=pltpu.CompilerParams(
            needs_layout_passes=False, use_tc_tiling_on_sc=False,
            disable_bounds_checks=True),
    )(_sc_body)

    x3 = x.reshape(R * NC, M)
    att3 = spatial_attention.reshape(C * NC, M)
    ck2 = lax.bitcast_convert_type(ck, jnp.int32).reshape(R, NC)
    w2 = conv_weight.reshape(C, K)
    bias2 = jnp.broadcast_to(conv_bias.reshape(C, 1), (C, L))
    locs, red = sc(x3, att3, ck2, w2, bias2)
    return (red[:, 0].reshape(B, C), locs.reshape(B, C, K))


# speedup vs baseline: 13.4898x; 1.0477x over previous
import functools

import jax
import jax.numpy as jnp
from jax import lax
from jax.experimental import pallas as pl
from jax.experimental.pallas import tpu as pltpu
from jax.experimental.pallas import tpu_sc as plsc

B, C, H, W = 16, 96, 224, 224
HW = H * W
R = B * C
M = 8
NC = HW // M
K = 1024
SUB = HW // 128
NSURV = 1280
NCAND = NSURV * M
NWIN = 1088
L = 16

_U32 = jnp.uint32


def _tc_cmax_kernel(x_ref, att_ref, out_ref):
    lidx = lax.broadcasted_iota(jnp.int32, (128, 16), 0)
    cidx = lax.broadcasted_iota(jnp.int32, (128, 16), 1)
    sel = (lidx == cidx * 8).astype(jnp.float32)
    att = att_ref[0]
    for q in range(8):
        a = x_ref[q, 0] * att
        m = a
        for s in (1, 2, 4):
            m = jnp.maximum(m, pltpu.roll(m, 128 - s, 1))
        cm = lax.dot_general(m, sel, (((1,), (0,)), ((), ())),
                             preferred_element_type=jnp.float32)
        bits = lax.bitcast_convert_type(cm, _U32)
        flip = jnp.where(bits >= _U32(0x80000000), _U32(0xFFFFFFFF),
                         _U32(0x80000000))
        out_ref[q, 0] = bits ^ flip


def _key16(v):
    bits = plsc.bitcast(v, _U32)
    flip = jnp.where(bits >= _U32(0x80000000), _U32(0xFFFFFFFF),
                     _U32(0x80000000))
    return bits ^ flip


def _unkey16(k):
    bits = jnp.where(k >= _U32(0x80000000), k ^ _U32(0x80000000), ~k)
    return plsc.bitcast(bits, jnp.float32)


def _sc_body(x_hbm, att_hbm, ck_hbm, w_hbm, bias_hbm, locs_hbm, red_hbm,
             ck_row, hist, offs, bnd_a, bnd_b, surv_x, surv_a,
             candx, canda, candk, wkk, wkp, skk, skp,
             wrow, biasv, outloc, outred, sem1, sem2):
    lanes = lax.iota(jnp.int32, L)
    ones = jnp.ones((L,), jnp.int32)
    zeros_i = jnp.zeros((L,), jnp.int32)

    def zero_hist(href):
        def zh(i, _):
            href[pl.ds(i * L, L)] = zeros_i
            return 0
        lax.fori_loop(0, 256 // L, zh, 0)

    def digits(k, shift):
        return lax.shift_right_logical(k, jnp.int32(shift)) & jnp.int32(0xFF)

    def hist_ref_pass(src, n, shift):
        zero_hist(hist)
        nb = (n + (4 * L - 1)) // (4 * L)

        def hp(i, _):
            base = i * (4 * L)
            for u in range(4):
                v = src[pl.ds(base + u * L, L)]
                valid = (lanes + (base + u * L)) < n
                plsc.addupdate_scatter(hist, [digits(v, shift)], ones,
                                       mask=valid)
            return 0
        lax.fori_loop(0, nb, hp, 0)

    def scan_hist(krem):
        def cond(carry):
            j, acc, found, beta, kr = carry
            return (found == 0) & (j < 256 // L)

        def sp(carry):
            j, acc, found, beta, kr = carry
            hv = hist[pl.ds(240 - L * j, L)]
            rv = lax.rev(hv, (0,))
            cs = plsc.cumsum(rv) + acc
            cross = cs >= krem
            anyc = jnp.max(cross.astype(jnp.int32))
            idx = jnp.max(plsc.all_reduce_ffs(cross))
            csv = jnp.max(jnp.where(lanes == idx, cs, 0))
            hvv = jnp.max(jnp.where(lanes == idx, rv, 0))
            nbeta = 255 - L * j - idx
            take = anyc == 1
            beta = jnp.where(take, nbeta, beta)
            kr = jnp.where(take, krem - (csv - hvv), kr)
            found = jnp.where(take, 1, found)
            return j + 1, jnp.max(cs), found, beta, kr
        z = jnp.int32(0)
        _, _, _, beta, kr = lax.while_loop(
            cond, sp, (z, z, z, z, jnp.int32(krem)))
        return beta, kr

    def compact_eq(src, n, shift, beta, dst):
        nb = (n + (4 * L - 1)) // (4 * L)

        def cp(i, cnt):
            base = i * (4 * L)
            for u in range(4):
                v = src[pl.ds(base + u * L, L)]
                valid = (lanes + (base + u * L)) < n
                m = valid & (digits(v, shift) == beta)
                cs = plsc.cumsum(m.astype(jnp.int32))
                plsc.store_scatter(dst, [cnt + cs - 1], v, mask=m)
                cnt = cnt + plsc.all_reduce_population_count(m)
            return cnt
        cntv = lax.fori_loop(0, nb, cp, jnp.zeros((L,), jnp.int32))
        return jnp.max(cntv)

    def radix_select(src, n, krem, skip_hist1=False):
        if not skip_hist1:
            hist_ref_pass(src, n, 24)
        b1, krem = scan_hist(krem)
        n1 = compact_eq(src, n, 24, b1, bnd_a)
        hist_ref_pass(bnd_a, n1, 16)
        b2, krem = scan_hist(krem)
        n2 = compact_eq(bnd_a, n1, 16, b2, bnd_b)
        hist_ref_pass(bnd_b, n2, 8)
        b3, krem = scan_hist(krem)
        n3 = compact_eq(bnd_b, n2, 8, b3, bnd_a)
        hist_ref_pass(bnd_a, n3, 0)
        b4, _ = scan_hist(krem)
        bu = lambda b, s: lax.shift_left(
            lax.convert_element_type(b, _U32), _U32(s))
        return bu(b1, 24) | bu(b2, 16) | bu(b3, 8) | bu(b4, 0)

    def radix_sort_pass(srck, srcp, dstk, dstp, shift):
        zero_hist(hist)

        def hp(i, _):
            base = i * (4 * L)
            for u in range(4):
                d = digits(srck[pl.ds(base + u * L, L)], shift)
                plsc.addupdate_scatter(hist, [d], ones)
            return 0
        lax.fori_loop(0, NWIN // (4 * L), hp, 0)

        def op(j, acc):
            base = j * (4 * L)
            for u in range(4):
                hv = hist[pl.ds(base + u * L, L)]
                cs = plsc.cumsum(hv) + acc
                offs[pl.ds(base + u * L, L)] = cs - hv
                acc = jnp.max(cs)
            return acc
        lax.fori_loop(0, 256 // (4 * L), op, jnp.int32(0))

        def mp(i, _):
            base = i * (4 * L)
            for u in range(4):
                kv = srck[pl.ds(base + u * L, L)]
                pv = srcp[pl.ds(base + u * L, L)]
                d = digits(kv, shift)
                cnt, _last = plsc.scan_count(plsc.bitcast(d, _U32))
                rank = plsc.bitcast(cnt, jnp.int32) - 1
                dest = plsc.load_gather(offs, [d]) + rank
                plsc.store_scatter(dstk, [dest], kv)
                plsc.store_scatter(dstp, [dest], pv)
                plsc.addupdate_scatter(offs, [d], ones)
            return 0
        lax.fori_loop(0, NWIN // (4 * L), mp, 0)

    wid = lax.axis_index("s") * 2 + lax.axis_index("c")
    nch = C // 32

    def ch_body(ci, _):
        ch = wid * nch + ci
        pltpu.sync_copy(w_hbm.at[ch], wrow)
        pltpu.sync_copy(bias_hbm.at[ch], biasv)

        def row_body(b, _):
            r = b * C + ch
            pltpu.sync_copy(ck_hbm.at[r], ck_row)

            t_c = radix_select(ck_row, NC, K)

            def pf(i, _):
                base = i * (4 * L)
                for u in range(4):
                    ids = lanes + base + u * L
                    surv_x[pl.ds(base + u * L, L)] = ids + r * NC
                    surv_a[pl.ds(base + u * L, L)] = ids + ch * NC
                return 0
            lax.fori_loop(0, NSURV // (4 * L), pf, 0)

            def sv(i, cnt):
                base = i * (4 * L)
                for u in range(4):
                    v = plsc.bitcast(ck_row[pl.ds(base + u * L, L)], _U32)
                    m = v >= t_c
                    cs = plsc.cumsum(m.astype(jnp.int32))
                    idx = cnt + cs - 1
                    m2 = m & (idx < NSURV)
                    ids = lanes + base + u * L
                    plsc.store_scatter(surv_x, [idx], ids + r * NC, mask=m2)
                    plsc.store_scatter(surv_a, [idx], ids + ch * NC, mask=m2)
                    cnt = cnt + plsc.all_reduce_population_count(m)
                return cnt
            n_s = jnp.max(lax.fori_loop(0, NC // (4 * L), sv,
                                        jnp.zeros((L,), jnp.int32)))
            n_s = jnp.minimum(n_s, NSURV)

            G = 128
            ntb = (n_s + (G - 1)) // G

            def gl(t, _):
                pltpu.make_async_copy(
                    x_hbm.at[surv_x.at[pl.ds(t * G, G)]],
                    candx.at[pl.ds(t * G, G)], sem1).start()
                pltpu.make_async_copy(
                    att_hbm.at[surv_a.at[pl.ds(t * G, G)]],
                    canda.at[pl.ds(t * G, G)], sem2).start()
                return 0
            lax.fori_loop(0, ntb, gl, 0)

            def gw(t, _):
                pltpu.make_async_copy(
                    x_hbm.at[pl.ds(0, G)],
                    candx.at[pl.ds(t * G, G)], sem1).wait()
                pltpu.make_async_copy(
                    att_hbm.at[pl.ds(0, G)],
                    canda.at[pl.ds(t * G, G)], sem2).wait()
                return 0
            lax.fori_loop(0, ntb, gw, 0)

            n_g = n_s * M
            zero_hist(hist)

            def kp(i, _):
                base = i * (4 * L)
                for u in range(4):
                    j = lanes + base + u * L
                    jc = j // M
                    jr = j - jc * M
                    xv = plsc.load_gather(candx, [jc, jr])
                    av = plsc.load_gather(canda, [jc, jr])
                    valid = j < n_g
                    kk = jnp.where(valid, _key16(xv * av), _U32(0))
                    ki = plsc.bitcast(kk, jnp.int32)
                    candk[pl.ds(base + u * L, L)] = ki
                    plsc.addupdate_scatter(hist, [digits(ki, 24)], ones,
                                           mask=valid)
                return 0
            lax.fori_loop(0, (n_g + (4 * L - 1)) // (4 * L), kp, 0)

            e_t = radix_select(candk, n_g, K, skip_hist1=True)

            neg1 = jnp.full((L,), -1, jnp.int32)

            def wf(i, _):
                base = i * (4 * L)
                for u in range(4):
                    wkk[pl.ds(base + u * L, L)] = neg1
                return 0
            lax.fori_loop(0, NWIN // (4 * L), wf, 0)

            def wp(i, cnt):
                base = i * (4 * L)
                for u in range(4):
                    j = lanes + base + u * L
                    kk = plsc.bitcast(candk[pl.ds(base + u * L, L)], _U32)
                    valid = j < n_g
                    m = valid & (kk >= e_t)
                    cs = plsc.cumsum(m.astype(jnp.int32))
                    idx = cnt + cs - 1
                    m2 = m & (idx < NWIN)
                    jc = j // M
                    gx = plsc.load_gather(surv_x, [jc]) - r * NC
                    pos = gx * M + (j - jc * M)
                    plsc.store_scatter(wkk, [idx],
                                       plsc.bitcast(~kk, jnp.int32), mask=m2)
                    plsc.store_scatter(wkp, [idx], pos, mask=m2)
                    cnt = cnt + plsc.all_reduce_population_count(m)
                return cnt
            lax.fori_loop(0, (n_g + (4 * L - 1)) // (4 * L), wp,
                          jnp.zeros((L,), jnp.int32))

            radix_sort_pass(wkk, wkp, skk, skp, 0)
            radix_sort_pass(skk, skp, wkk, wkp, 8)
            radix_sort_pass(wkk, wkp, skk, skp, 16)
            radix_sort_pass(skk, skp, wkk, wkp, 24)

            def rp(i, acc):
                base = i * (4 * L)
                for u in range(4):
                    inv = plsc.bitcast(wkk[pl.ds(base + u * L, L)], _U32)
                    p = wkp[pl.ds(base + u * L, L)]
                    val = _unkey16(~inv)
                    wv = wrow[pl.ds(base + u * L, L)]
                    outloc[pl.ds(base + u * L, L)] = lax.convert_element_type(
                        p, jnp.float32)
                    acc = acc + val * wv
                return acc
            acc = lax.fori_loop(0, K // (4 * L), rp,
                                jnp.zeros((L,), jnp.float32))
            red = jnp.sum(acc) + jnp.max(biasv[...])
            outred[...] = jnp.broadcast_to(red, (L,))
            pltpu.sync_copy(outloc, locs_hbm.at[r])
            pltpu.sync_copy(outred, red_hbm.at[r])
            return 0
        lax.fori_loop(0, B, row_body, 0)
        return 0
    lax.fori_loop(0, nch, ch_body, 0)


def kernel(x, spatial_attention, conv_weight, conv_bias):
    x4 = x.reshape(B, C, SUB, 128)
    att4 = spatial_attention.reshape(C, SUB, 128)
    ck = pl.pallas_call(
        _tc_cmax_kernel,
        grid=(C, B // 8),
        in_specs=[
            pl.BlockSpec((8, 1, SUB, 128), lambda c, b: (b, c, 0, 0)),
            pl.BlockSpec((1, SUB, 128), lambda c, b: (c, 0, 0)),
        ],
        out_specs=pl.BlockSpec((8, 1, SUB, 16), lambda c, b: (b, c, 0, 0)),
        out_shape=jax.ShapeDtypeStruct((B, C, SUB, 16), _U32),
    )(x4, att4)

    mesh = plsc.VectorSubcoreMesh(core_axis_name="c", subcore_axis_name="s")
    sc = functools.partial(
        pl.kernel,
        out_type=(jax.ShapeDtypeStruct((R, K), jnp.float32),
                  jax.ShapeDtypeStruct((R, L), jnp.float32)),
        mesh=mesh,
        scratch_types=[
            pltpu.VMEM((NC,), jnp.int32),
            pltpu.VMEM((256,), jnp.int32),
            pltpu.VMEM((256,), jnp.int32),
            pltpu.VMEM((NCAND,), jnp.int32),
            pltpu.VMEM((NCAND,), jnp.int32),
            pltpu.VMEM((NSURV,), jnp.int32),
            pltpu.VMEM((NSURV,), jnp.int32),
            pltpu.VMEM((NSURV, M), jnp.float32),
            pltpu.VMEM((NSURV, M), jnp.float32),
            pltpu.VMEM((NCAND,), jnp.int32),
            pltpu.VMEM((NWIN,), jnp.int32),
            pltpu.VMEM((NWIN,), jnp.int32),
            pltpu.VMEM((NWIN,), jnp.int32),
            pltpu.VMEM((NWIN,), jnp.int32),
            pltpu.VMEM((K,), jnp.float32),
            pltpu.VMEM((L,), jnp.float32),
            pltpu.VMEM((K,), jnp.float32),
            pltpu.VMEM((L,), jnp.float32),
            pltpu.SemaphoreType.DMA,
            pltpu.SemaphoreType.DMA,
        ],
        compiler_params=pltpu.CompilerParams(
            needs_layout_passes=False, use_tc_tiling_on_sc=False,
            disable_bounds_checks=True),
    )(_sc_body)

    x3 = x.reshape(R * NC, M)
    att3 = spatial_attention.reshape(C * NC, M)
    ck2 = lax.bitcast_convert_type(ck, jnp.int32).reshape(R, NC)
    w2 = conv_weight.reshape(C, K)
    bias2 = jnp.broadcast_to(conv_bias.reshape(C, 1), (C, L))
    locs, red = sc(x3, att3, ck2, w2, bias2)
    return (red[:, 0].reshape(B, C), locs.reshape(B, C, K))
